# baseline (speedup 1.0000x reference)
import functools

import jax
import jax.numpy as jnp
import numpy as np
from jax import lax
from jax.experimental import pallas as pl
from jax.experimental.pallas import tpu as pltpu
from jax.experimental.pallas import tpu_sc as plsc

_LEVELS = 16
_FEAT = 2
_LOG2_T = 19
_T = 1 << _LOG2_T
_HASH_MASK = _T - 1
_N = 131072
_HIDDEN = 256
_BETA = 100.0
_MIN_LOGRES, _MAX_LOGRES = 4, 11
_R_MIN, _R_MAX = 2 ** _MIN_LOGRES, 2 ** _MAX_LOGRES
_GROWTH = np.exp((np.log(_R_MAX) - np.log(_R_MIN)) / (_LEVELS - 1))
_RES = [int(np.floor(_R_MIN * _GROWTH ** lv)) for lv in range(_LEVELS)]
_P2 = np.int32(np.uint32(2654435761).view(np.int32))
_P3 = np.int32(805459861)

_NC, _NS = 2, 16
_NW = _NC * _NS
_PER_W = _N // _NW
_P = 1024
_NCHUNK = _PER_W // _P
_G = _P // 16


def _encode_body(pts_hbm, tab0_hbm, tab1_hbm, enc_hbm,
                 pts_v, idx_a, idx_b, vals_a, vals_b, enc_v, sem_a, sem_b):
    wid = lax.axis_index("s") * _NC + lax.axis_index("c")
    wbase = wid * _PER_W
    lane = lax.iota(jnp.int32, 16)

    def chunk_body(ci, carry):
        cbase = wbase + ci * _P
        for d in range(3):
            pltpu.sync_copy(pts_hbm.at[pl.ds(d * _N + cbase, _P)],
                            pts_v.at[pl.ds(d * _P, _P)])

        def hash_fire(lv, idx_v, vals_v, sem):
            res = float(_RES[lv])
            lvoff = lv * _T

            def grp_hash(g, c2):
                s = g * 16
                px = pts_v[pl.ds(s, 16)]
                py = pts_v[pl.ds(_P + s, 16)]
                pz = pts_v[pl.ds(2 * _P + s, 16)]
                x = ((px + 1.0) * 0.5) * res
                y = ((py + 1.0) * 0.5) * res
                z = ((pz + 1.0) * 0.5) * res
                ix = x.astype(jnp.int32)
                iy = y.astype(jnp.int32)
                iz = z.astype(jnp.int32)
                ix1 = ix + 1
                my0 = iy * _P2
                my1 = my0 + _P2
                mz0 = iz * _P3
                mz1 = mz0 + _P3
                b = g * 128
                for c in range(8):
                    hx = ix1 if (c >> 2) & 1 else ix
                    hy = my1 if (c >> 1) & 1 else my0
                    hz = mz1 if c & 1 else mz0
                    w0 = ((hx ^ hy ^ hz) & _HASH_MASK) + lvoff
                    idx_v[pl.ds(b + c * 16, 16)] = w0
                pltpu.async_copy(
                    tab0_hbm.at[idx_v.at[pl.ds(b, 128)]],
                    vals_v.at[pl.ds(b, 128)],
                    sem,
                )
                pltpu.async_copy(
                    tab1_hbm.at[idx_v.at[pl.ds(b, 128)]],
                    vals_v.at[pl.ds(8 * _P + b, 128)],
                    sem,
                )
                return c2

            lax.fori_loop(0, _G, grp_hash, 0)

        def drain(idx_v, vals_v, sem):
            def drain_j(j, c2):
                b = j * 128
                pltpu.make_async_copy(
                    tab0_hbm.at[idx_v.at[pl.ds(b, 128)]],
                    vals_v.at[pl.ds(b, 128)],
                    sem,
                ).wait()
                pltpu.make_async_copy(
                    tab1_hbm.at[idx_v.at[pl.ds(b, 128)]],
                    vals_v.at[pl.ds(8 * _P + b, 128)],
                    sem,
                ).wait()
                return c2

            lax.fori_loop(0, _G, drain_j, 0)

        def interp(lv, vals_v):
            res = float(_RES[lv])

            def grp_interp(g, c2):
                s = g * 16
                px = pts_v[pl.ds(s, 16)]
                py = pts_v[pl.ds(_P + s, 16)]
                pz = pts_v[pl.ds(2 * _P + s, 16)]
                x = ((px + 1.0) * 0.5) * res
                y = ((py + 1.0) * 0.5) * res
                z = ((pz + 1.0) * 0.5) * res
                fx = x - x.astype(jnp.int32).astype(jnp.float32)
                fy = y - y.astype(jnp.int32).astype(jnp.float32)
                fz = z - z.astype(jnp.int32).astype(jnp.float32)
                gx = 1.0 - fx
                gy = 1.0 - fy
                gz = 1.0 - fz
                acc0 = jnp.zeros((16,), jnp.float32)
                acc1 = jnp.zeros((16,), jnp.float32)
                b = g * 128
                for c in range(8):
                    wx = fx if (c >> 2) & 1 else gx
                    wy = fy if (c >> 1) & 1 else gy
                    wz = fz if c & 1 else gz
                    w = wx * wy * wz
                    v0 = vals_v[pl.ds(b + c * 16, 16)]
                    v1 = vals_v[pl.ds(8 * _P + b + c * 16, 16)]
                    acc0 = acc0 + w * v0
                    acc1 = acc1 + w * v1
                enc_v[2 * lv, pl.ds(s, 16)] = acc0
                enc_v[2 * lv + 1, pl.ds(s, 16)] = acc1
                return c2

            lax.fori_loop(0, _G, grp_interp, 0)

        bufs = ((idx_a, vals_a, sem_a), (idx_b, vals_b, sem_b))
        hash_fire(0, *bufs[0])
        for lv in range(_LEVELS):
            cur = bufs[lv % 2]
            if lv + 1 < _LEVELS:
                hash_fire(lv + 1, *bufs[(lv + 1) % 2])
            drain(*cur)
            interp(lv, cur[1])

        pltpu.sync_copy(enc_v, enc_hbm.at[:, pl.ds(cbase, _P)])
        return carry

    lax.fori_loop(0, _NCHUNK, chunk_body, 0)


_enc_kernel = functools.partial(
    pl.kernel,
    out_type=jax.ShapeDtypeStruct((2 * _LEVELS, _N), jnp.float32),
    mesh=plsc.VectorSubcoreMesh(core_axis_name="c", subcore_axis_name="s"),
    scratch_types=[
        pltpu.VMEM((3 * _P,), jnp.float32),
        pltpu.VMEM((_P * 8,), jnp.int32),
        pltpu.VMEM((_P * 8,), jnp.int32),
        pltpu.VMEM((_P * 16,), jnp.float32),
        pltpu.VMEM((_P * 16,), jnp.float32),
        pltpu.VMEM((2 * _LEVELS, _P), jnp.float32),
        pltpu.SemaphoreType.DMA,
        pltpu.SemaphoreType.DMA,
    ],
)(_encode_body)


_BN = 512


def _softplus_beta(z):
    t = _BETA * z
    sp = jnp.maximum(t, 0.0) + jnp.log1p(jnp.exp(-jnp.abs(t)))
    return sp * (1.0 / _BETA)


def _mlp_body(x3_ref, enc_ref, w0p_ref, w0e_ref, b0_ref,
              w1_ref, b1_ref, w2s_ref, b2s_ref, w2f_ref, b2f_ref,
              sdf_ref, feat_ref):
    z0 = (jnp.dot(x3_ref[...], w0p_ref[...], preferred_element_type=jnp.float32)
          + lax.dot_general(enc_ref[...], w0e_ref[...],
                            dimension_numbers=(((0,), (0,)), ((), ())),
                            preferred_element_type=jnp.float32)
          + b0_ref[...])
    h0 = _softplus_beta(z0).astype(jnp.bfloat16)
    z1 = jnp.dot(h0, w1_ref[...], preferred_element_type=jnp.float32) + b1_ref[...]
    h1 = _softplus_beta(z1).astype(jnp.bfloat16)
    sdf_ref[...] = (jnp.dot(h1, w2s_ref[...], preferred_element_type=jnp.float32)
                    + b2s_ref[...])
    feat_ref[...] = (jnp.dot(h1, w2f_ref[...], preferred_element_type=jnp.float32)
                     + b2f_ref[...])


def _mlp(x3, enc, w0p, w0e, b0, w1, b1, w2s, b2s, w2f, b2f):
    grid = _N // _BN
    return pl.pallas_call(
        _mlp_body,
        grid=(grid,),
        in_specs=[
            pl.BlockSpec((_BN, 3), lambda i: (i, 0)),
            pl.BlockSpec((2 * _LEVELS, _BN), lambda i: (0, i)),
            pl.BlockSpec((3, _HIDDEN), lambda i: (0, 0)),
            pl.BlockSpec((2 * _LEVELS, _HIDDEN), lambda i: (0, 0)),
            pl.BlockSpec((1, _HIDDEN), lambda i: (0, 0)),
            pl.BlockSpec((_HIDDEN, _HIDDEN), lambda i: (0, 0)),
            pl.BlockSpec((1, _HIDDEN), lambda i: (0, 0)),
            pl.BlockSpec((_HIDDEN, 1), lambda i: (0, 0)),
            pl.BlockSpec((1, 1), lambda i: (0, 0)),
            pl.BlockSpec((_HIDDEN, _HIDDEN - 1), lambda i: (0, 0)),
            pl.BlockSpec((1, _HIDDEN - 1), lambda i: (0, 0)),
        ],
        out_specs=[
            pl.BlockSpec((_BN, 1), lambda i: (i, 0)),
            pl.BlockSpec((_BN, _HIDDEN - 1), lambda i: (i, 0)),
        ],
        out_shape=[
            jax.ShapeDtypeStruct((_N, 1), jnp.float32),
            jax.ShapeDtypeStruct((_N, _HIDDEN - 1), jnp.float32),
        ],
    )(x3, enc, w0p, w0e, b0, w1, b1, w2s, b2s, w2f, b2f)


def kernel(points_3D, table, W0, b0, W1, b1, W2, b2):
    c1 = lax.optimization_barrier(jnp.float32(1.0))
    pts_flat = points_3D.T.reshape(3 * _N) * c1
    tab0 = table[:, :, 0].reshape(_LEVELS * _T) * c1
    tab1 = table[:, :, 1].reshape(_LEVELS * _T) * c1
    enc_t = _enc_kernel(pts_flat, tab0, tab1)
    bf = jnp.bfloat16
    sdf, feat = _mlp(points_3D, enc_t,
                     W0[:3], W0[3:], b0.reshape(1, _HIDDEN),
                     W1.astype(bf), b1.reshape(1, _HIDDEN),
                     W2[:, :1].astype(bf), b2[:1].reshape(1, 1),
                     W2[:, 1:].astype(bf), b2[1:].reshape(1, _HIDDEN - 1))
    return (sdf, feat)


# speedup vs baseline: 1.0132x; 1.0132x over previous
import functools

import jax
import jax.numpy as jnp
import numpy as np
from jax import lax
from jax.experimental import pallas as pl
from jax.experimental.pallas import tpu as pltpu
from jax.experimental.pallas import tpu_sc as plsc

_LEVELS = 16
_FEAT = 2
_LOG2_T = 19
_T = 1 << _LOG2_T
_HASH_MASK = _T - 1
_N = 131072
_HIDDEN = 256
_BETA = 100.0
_MIN_LOGRES, _MAX_LOGRES = 4, 11
_R_MIN, _R_MAX = 2 ** _MIN_LOGRES, 2 ** _MAX_LOGRES
_GROWTH = np.exp((np.log(_R_MAX) - np.log(_R_MIN)) / (_LEVELS - 1))
_RES = [int(np.floor(_R_MIN * _GROWTH ** lv)) for lv in range(_LEVELS)]
_P2 = np.int32(np.uint32(2654435761).view(np.int32))
_P3 = np.int32(805459861)

_NC, _NS = 2, 16
_NW = _NC * _NS
_PER_W = _N // _NW
_P = 1024
_NCHUNK = _PER_W // _P
_G = _P // 16


def _encode_body(pts_hbm, tab0_hbm, tab1_hbm, enc_hbm,
                 pts_v, idx_a, idx_b, vals_a, vals_b, enc_v, sem_a, sem_b):
    wid = lax.axis_index("s") * _NC + lax.axis_index("c")
    wbase = wid * _PER_W
    lane = lax.iota(jnp.int32, 16)

    def chunk_body(ci, carry):
        cbase = wbase + ci * _P
        for d in range(3):
            pltpu.sync_copy(pts_hbm.at[pl.ds(d * _N + cbase, _P)],
                            pts_v.at[pl.ds(d * _P, _P)])

        def hash_fire(lv, idx_v, vals_v, sem):
            res = float(_RES[lv])
            lvoff = lv * _T

            def grp_hash(g, c2):
                s = g * 16
                px = pts_v[pl.ds(s, 16)]
                py = pts_v[pl.ds(_P + s, 16)]
                pz = pts_v[pl.ds(2 * _P + s, 16)]
                x = ((px + 1.0) * 0.5) * res
                y = ((py + 1.0) * 0.5) * res
                z = ((pz + 1.0) * 0.5) * res
                ix = x.astype(jnp.int32)
                iy = y.astype(jnp.int32)
                iz = z.astype(jnp.int32)
                ix1 = ix + 1
                my0 = iy * _P2
                my1 = my0 + _P2
                mz0 = iz * _P3
                mz1 = mz0 + _P3
                b = g * 128
                for c in range(8):
                    hx = ix1 if (c >> 2) & 1 else ix
                    hy = my1 if (c >> 1) & 1 else my0
                    hz = mz1 if c & 1 else mz0
                    w0 = ((hx ^ hy ^ hz) & _HASH_MASK) + lvoff
                    idx_v[pl.ds(b + c * 16, 16)] = w0
                pltpu.async_copy(
                    tab0_hbm.at[idx_v.at[pl.ds(b, 128)]],
                    vals_v.at[pl.ds(b, 128)],
                    sem,
                )
                pltpu.async_copy(
                    tab1_hbm.at[idx_v.at[pl.ds(b, 128)]],
                    vals_v.at[pl.ds(8 * _P + b, 128)],
                    sem,
                )
                return c2

            lax.fori_loop(0, _G, grp_hash, 0)

        def drain(idx_v, vals_v, sem):
            def drain_j(j, c2):
                b = j * 128
                pltpu.make_async_copy(
                    tab0_hbm.at[idx_v.at[pl.ds(b, 128)]],
                    vals_v.at[pl.ds(b, 128)],
                    sem,
                ).wait()
                pltpu.make_async_copy(
                    tab1_hbm.at[idx_v.at[pl.ds(b, 128)]],
                    vals_v.at[pl.ds(8 * _P + b, 128)],
                    sem,
                ).wait()
                return c2

            lax.fori_loop(0, _G, drain_j, 0)

        def interp(lv, vals_v):
            res = float(_RES[lv])

            def grp_interp(g, c2):
                s = g * 16
                px = pts_v[pl.ds(s, 16)]
                py = pts_v[pl.ds(_P + s, 16)]
                pz = pts_v[pl.ds(2 * _P + s, 16)]
                x = ((px + 1.0) * 0.5) * res
                y = ((py + 1.0) * 0.5) * res
                z = ((pz + 1.0) * 0.5) * res
                fx = x - x.astype(jnp.int32).astype(jnp.float32)
                fy = y - y.astype(jnp.int32).astype(jnp.float32)
                fz = z - z.astype(jnp.int32).astype(jnp.float32)
                gx = 1.0 - fx
                gy = 1.0 - fy
                gz = 1.0 - fz
                acc0 = jnp.zeros((16,), jnp.float32)
                acc1 = jnp.zeros((16,), jnp.float32)
                b = g * 128
                for c in range(8):
                    wx = fx if (c >> 2) & 1 else gx
                    wy = fy if (c >> 1) & 1 else gy
                    wz = fz if c & 1 else gz
                    w = wx * wy * wz
                    v0 = vals_v[pl.ds(b + c * 16, 16)]
                    v1 = vals_v[pl.ds(8 * _P + b + c * 16, 16)]
                    acc0 = acc0 + w * v0
                    acc1 = acc1 + w * v1
                enc_v[2 * lv, pl.ds(s, 16)] = acc0
                enc_v[2 * lv + 1, pl.ds(s, 16)] = acc1
                return c2

            lax.fori_loop(0, _G, grp_interp, 0)

        bufs = ((idx_a, vals_a, sem_a), (idx_b, vals_b, sem_b))
        hash_fire(0, *bufs[0])
        for lv in range(_LEVELS):
            cur = bufs[lv % 2]
            if lv + 1 < _LEVELS:
                hash_fire(lv + 1, *bufs[(lv + 1) % 2])
            drain(*cur)
            interp(lv, cur[1])

        pltpu.sync_copy(enc_v, enc_hbm.at[:, pl.ds(cbase, _P)])
        return carry

    lax.fori_loop(0, _NCHUNK, chunk_body, 0)


_enc_kernel = functools.partial(
    pl.kernel,
    out_type=jax.ShapeDtypeStruct((2 * _LEVELS, _N), jnp.float32),
    mesh=plsc.VectorSubcoreMesh(core_axis_name="c", subcore_axis_name="s"),
    scratch_types=[
        pltpu.VMEM((3 * _P,), jnp.float32),
        pltpu.VMEM((_P * 8,), jnp.int32),
        pltpu.VMEM((_P * 8,), jnp.int32),
        pltpu.VMEM((_P * 16,), jnp.float32),
        pltpu.VMEM((_P * 16,), jnp.float32),
        pltpu.VMEM((2 * _LEVELS, _P), jnp.float32),
        pltpu.SemaphoreType.DMA,
        pltpu.SemaphoreType.DMA,
    ],
)(_encode_body)


_BN = 512


def _softplus_beta(z):
    t = _BETA * z
    sp = jnp.maximum(t, 0.0) + jnp.log1p(jnp.exp(-jnp.abs(t)))
    return sp * (1.0 / _BETA)


def _mlp_body(x3_ref, enc_ref, w0p_ref, w0e_ref, b0_ref,
              w1_ref, b1_ref, w2s_ref, b2s_ref, w2f_ref, b2f_ref,
              sdf_ref, feat_ref):
    z0 = (jnp.dot(x3_ref[...], w0p_ref[...], preferred_element_type=jnp.float32)
          + lax.dot_general(enc_ref[...], w0e_ref[...],
                            dimension_numbers=(((0,), (0,)), ((), ())),
                            preferred_element_type=jnp.float32)
          + b0_ref[...])
    h0 = _softplus_beta(z0)
    z1 = jnp.dot(h0, w1_ref[...], preferred_element_type=jnp.float32) + b1_ref[...]
    h1 = _softplus_beta(z1)
    sdf_ref[...] = (jnp.dot(h1, w2s_ref[...], preferred_element_type=jnp.float32)
                    + b2s_ref[...])
    feat_ref[...] = (jnp.dot(h1, w2f_ref[...], preferred_element_type=jnp.float32)
                     + b2f_ref[...])


def _mlp(x3, enc, w0p, w0e, b0, w1, b1, w2s, b2s, w2f, b2f):
    grid = _N // _BN
    return pl.pallas_call(
        _mlp_body,
        grid=(grid,),
        in_specs=[
            pl.BlockSpec((_BN, 3), lambda i: (i, 0)),
            pl.BlockSpec((2 * _LEVELS, _BN), lambda i: (0, i)),
            pl.BlockSpec((3, _HIDDEN), lambda i: (0, 0)),
            pl.BlockSpec((2 * _LEVELS, _HIDDEN), lambda i: (0, 0)),
            pl.BlockSpec((1, _HIDDEN), lambda i: (0, 0)),
            pl.BlockSpec((_HIDDEN, _HIDDEN), lambda i: (0, 0)),
            pl.BlockSpec((1, _HIDDEN), lambda i: (0, 0)),
            pl.BlockSpec((_HIDDEN, 1), lambda i: (0, 0)),
            pl.BlockSpec((1, 1), lambda i: (0, 0)),
            pl.BlockSpec((_HIDDEN, _HIDDEN - 1), lambda i: (0, 0)),
            pl.BlockSpec((1, _HIDDEN - 1), lambda i: (0, 0)),
        ],
        out_specs=[
            pl.BlockSpec((_BN, 1), lambda i: (i, 0)),
            pl.BlockSpec((_BN, _HIDDEN - 1), lambda i: (i, 0)),
        ],
        out_shape=[
            jax.ShapeDtypeStruct((_N, 1), jnp.float32),
            jax.ShapeDtypeStruct((_N, _HIDDEN - 1), jnp.float32),
        ],
    )(x3, enc, w0p, w0e, b0, w1, b1, w2s, b2s, w2f, b2f)


def kernel(points_3D, table, W0, b0, W1, b1, W2, b2):
    c1 = lax.optimization_barrier(jnp.float32(1.0))
    pts_flat = points_3D.T.reshape(3 * _N) * c1
    tab0 = table[:, :, 0].reshape(_LEVELS * _T) * c1
    tab1 = table[:, :, 1].reshape(_LEVELS * _T) * c1
    enc_t = _enc_kernel(pts_flat, tab0, tab1)
    sdf, feat = _mlp(points_3D, enc_t,
                     W0[:3], W0[3:], b0.reshape(1, _HIDDEN),
                     W1, b1.reshape(1, _HIDDEN),
                     W2[:, :1], b2[:1].reshape(1, 1),
                     W2[:, 1:], b2[1:].reshape(1, _HIDDEN - 1))
    return (sdf, feat)
